# baseline (speedup 1.0000x reference)
import functools

import jax
import jax.numpy as jnp
from jax import lax
from jax.experimental import pallas as pl
from jax.experimental.pallas import tpu as pltpu
from jax.experimental.pallas import tpu_sc as plsc

L = 16
NC = 2
NS = 16
NW = NC * NS
S = 8
U = 8


def _body(n_points, qpw, values_hbm, t_hbm, out_hbm,
          t_v, idx0_v, idx1_v, v0_v, v1_v, out_v, sem0, sem1):
    wid = lax.axis_index("s") * NC + lax.axis_index("c")
    base = wid * qpw
    hi = jnp.int32(n_points - 1)
    one = jnp.int32(1)
    qps = qpw // S

    pltpu.sync_copy(t_hbm.at[pl.ds(base, qpw)], t_v)

    def idx_row(j, carry):
        for k in range(U):
            o = j * (U * L) + k * L
            tv = t_v[pl.ds(o, L)]
            idx = jnp.clip(tv.astype(jnp.int32) + one, one, hi)
            idx0_v[pl.ds(o, L)] = idx - one
            idx1_v[pl.ds(o, L)] = idx
        return carry

    def lerp_row(j, carry):
        for k in range(U):
            o = j * (U * L) + k * L
            tv = t_v[pl.ds(o, L)]
            idx = jnp.clip(tv.astype(jnp.int32) + one, one, hi)
            t0 = (idx - one).astype(jnp.float32)
            v0 = v0_v[pl.ds(o, L)]
            v1 = v1_v[pl.ds(o, L)]
            out_v[pl.ds(o, L)] = v0 + (tv - t0) * (v1 - v0)
        return carry

    rows_per_slice = qps // (U * L)

    for s_ in range(S):
        lo = s_ * qps
        lax.fori_loop(s_ * rows_per_slice, (s_ + 1) * rows_per_slice,
                      idx_row, 0)
        pltpu.async_copy(values_hbm.at[idx0_v.at[pl.ds(lo, qps)]],
                         v0_v.at[pl.ds(lo, qps)], sem0)
        pltpu.async_copy(values_hbm.at[idx1_v.at[pl.ds(lo, qps)]],
                         v1_v.at[pl.ds(lo, qps)], sem1)

    for s_ in range(S):
        lo = s_ * qps
        pltpu.make_async_copy(values_hbm.at[idx0_v.at[pl.ds(lo, qps)]],
                              v0_v.at[pl.ds(lo, qps)], sem0).wait()
        pltpu.make_async_copy(values_hbm.at[idx1_v.at[pl.ds(lo, qps)]],
                              v1_v.at[pl.ds(lo, qps)], sem1).wait()
        lax.fori_loop(s_ * rows_per_slice, (s_ + 1) * rows_per_slice,
                      lerp_row, 0)

    pltpu.sync_copy(out_v, out_hbm.at[pl.ds(base, qpw)])


@jax.jit
def kernel(times, values, t):
    nq = t.shape[0]
    qpw = nq // NW
    mesh = plsc.VectorSubcoreMesh(core_axis_name="c", subcore_axis_name="s")
    f = pl.kernel(
        functools.partial(_body, times.shape[0], qpw),
        out_type=jax.ShapeDtypeStruct((nq,), jnp.float32),
        mesh=mesh,
        scratch_types=[
            pltpu.VMEM((qpw,), jnp.float32),
            pltpu.VMEM((qpw,), jnp.int32),
            pltpu.VMEM((qpw,), jnp.int32),
            pltpu.VMEM((qpw,), jnp.float32),
            pltpu.VMEM((qpw,), jnp.float32),
            pltpu.VMEM((qpw,), jnp.float32),
            pltpu.SemaphoreType.DMA,
            pltpu.SemaphoreType.DMA,
        ],
    )
    return f(values, t)


# speedup vs baseline: 2.2546x; 2.2546x over previous
import functools

import jax
import jax.numpy as jnp
from jax import lax
from jax.experimental import pallas as pl
from jax.experimental.pallas import tpu as pltpu
from jax.experimental.pallas import tpu_sc as plsc

L = 16
NC = 2
NS = 16
NW = NC * NS
U = 8


def _body(n_points, qpw, values_hbm, t_hbm, out_hbm,
          values_sp, t_v, idx0_v, idx1_v, v0_v, v1_v, out_v, sem0, sem1):
    sid = lax.axis_index("s")
    wid = sid * NC + lax.axis_index("c")
    base = wid * qpw
    hi = jnp.int32(n_points - 1)
    one = jnp.int32(1)

    vps = n_points // NS
    pltpu.sync_copy(values_hbm.at[pl.ds(sid * vps, vps)],
                    values_sp.at[pl.ds(sid * vps, vps)])

    pltpu.sync_copy(t_hbm.at[pl.ds(base, qpw)], t_v)

    def idx_row(j, carry):
        for k in range(U):
            o = j * (U * L) + k * L
            tv = t_v[pl.ds(o, L)]
            idx = jnp.clip(tv.astype(jnp.int32) + one, one, hi)
            idx0_v[pl.ds(o, L)] = idx - one
            idx1_v[pl.ds(o, L)] = idx
        return carry

    lax.fori_loop(0, qpw // (U * L), idx_row, 0)

    plsc.subcore_barrier()

    pltpu.async_copy(values_sp.at[idx0_v], v0_v, sem0)
    pltpu.async_copy(values_sp.at[idx1_v], v1_v, sem1)
    pltpu.make_async_copy(values_sp.at[idx0_v], v0_v, sem0).wait()
    pltpu.make_async_copy(values_sp.at[idx1_v], v1_v, sem1).wait()

    def lerp_row(j, carry):
        for k in range(U):
            o = j * (U * L) + k * L
            tv = t_v[pl.ds(o, L)]
            idx = jnp.clip(tv.astype(jnp.int32) + one, one, hi)
            t0 = (idx - one).astype(jnp.float32)
            v0 = v0_v[pl.ds(o, L)]
            v1 = v1_v[pl.ds(o, L)]
            out_v[pl.ds(o, L)] = v0 + (tv - t0) * (v1 - v0)
        return carry

    lax.fori_loop(0, qpw // (U * L), lerp_row, 0)

    pltpu.sync_copy(out_v, out_hbm.at[pl.ds(base, qpw)])


@jax.jit
def kernel(times, values, t):
    nq = t.shape[0]
    qpw = nq // NW
    mesh = plsc.VectorSubcoreMesh(core_axis_name="c", subcore_axis_name="s")
    f = pl.kernel(
        functools.partial(_body, times.shape[0], qpw),
        out_type=jax.ShapeDtypeStruct((nq,), jnp.float32),
        mesh=mesh,
        scratch_types=[
            pltpu.VMEM_SHARED((times.shape[0],), jnp.float32),
            pltpu.VMEM((qpw,), jnp.float32),
            pltpu.VMEM((qpw,), jnp.int32),
            pltpu.VMEM((qpw,), jnp.int32),
            pltpu.VMEM((qpw,), jnp.float32),
            pltpu.VMEM((qpw,), jnp.float32),
            pltpu.VMEM((qpw,), jnp.float32),
            pltpu.SemaphoreType.DMA,
            pltpu.SemaphoreType.DMA,
        ],
    )
    return f(values, t)


# speedup vs baseline: 2.4376x; 1.0812x over previous
import functools

import jax
import jax.numpy as jnp
from jax import lax
from jax.experimental import pallas as pl
from jax.experimental.pallas import tpu as pltpu
from jax.experimental.pallas import tpu_sc as plsc

L = 16
NC = 2
NS = 16
NW = NC * NS
U = 8


def _body(n_points, qpw, values_hbm, t_hbm, out_hbm,
          values_sp, t_v, idx0_v, idx1_v, v0_v, v1_v, out_v, sem0, sem1):
    sid = lax.axis_index("s")
    wid = sid * NC + lax.axis_index("c")
    base = wid * qpw
    hi = jnp.int32(n_points - 1)
    one = jnp.int32(1)

    vps = n_points // NS
    stage = pltpu.async_copy(values_hbm.at[pl.ds(sid * vps, vps)],
                             values_sp.at[pl.ds(sid * vps, vps)], sem0)

    pltpu.sync_copy(t_hbm.at[pl.ds(base, qpw)], t_v)

    def idx_row(j, carry):
        for k in range(U):
            o = j * (U * L) + k * L
            tv = t_v[pl.ds(o, L)]
            idx = jnp.clip(tv.astype(jnp.int32) + one, one, hi)
            idx0 = idx - one
            idx0_v[pl.ds(o, L)] = idx0
            idx1_v[pl.ds(o, L)] = idx
            t_v[pl.ds(o, L)] = tv - idx0.astype(jnp.float32)
        return carry

    lax.fori_loop(0, qpw // (U * L), idx_row, 0)

    stage.wait()
    plsc.subcore_barrier()

    pltpu.async_copy(values_sp.at[idx0_v], v0_v, sem0)
    pltpu.async_copy(values_sp.at[idx1_v], v1_v, sem1)
    pltpu.make_async_copy(values_sp.at[idx0_v], v0_v, sem0).wait()
    pltpu.make_async_copy(values_sp.at[idx1_v], v1_v, sem1).wait()

    def lerp_row(j, carry):
        for k in range(U):
            o = j * (U * L) + k * L
            frac = t_v[pl.ds(o, L)]
            v0 = v0_v[pl.ds(o, L)]
            v1 = v1_v[pl.ds(o, L)]
            out_v[pl.ds(o, L)] = v0 + frac * (v1 - v0)
        return carry

    lax.fori_loop(0, qpw // (U * L), lerp_row, 0)

    pltpu.sync_copy(out_v, out_hbm.at[pl.ds(base, qpw)])


@jax.jit
def kernel(times, values, t):
    nq = t.shape[0]
    qpw = nq // NW
    mesh = plsc.VectorSubcoreMesh(core_axis_name="c", subcore_axis_name="s")
    f = pl.kernel(
        functools.partial(_body, times.shape[0], qpw),
        out_type=jax.ShapeDtypeStruct((nq,), jnp.float32),
        mesh=mesh,
        scratch_types=[
            pltpu.VMEM_SHARED((times.shape[0],), jnp.float32),
            pltpu.VMEM((qpw,), jnp.float32),
            pltpu.VMEM((qpw,), jnp.int32),
            pltpu.VMEM((qpw,), jnp.int32),
            pltpu.VMEM((qpw,), jnp.float32),
            pltpu.VMEM((qpw,), jnp.float32),
            pltpu.VMEM((qpw,), jnp.float32),
            pltpu.SemaphoreType.DMA,
            pltpu.SemaphoreType.DMA,
        ],
    )
    return f(values, t)
